# Initial kernel scaffold; baseline (speedup 1.0000x reference)
import jax
import jax.numpy as jnp
from jax.experimental import pallas as pl


def kernel(x, edge_index, join_index, W1, b1, W2, b2, W3, b3):
    raise NotImplementedError("write your pallas kernel here")


# speedup vs baseline: 13.1296x; 13.1296x over previous
import functools

import jax
import jax.numpy as jnp
from jax import lax
from jax.experimental import pallas as pl
from jax.experimental.pallas import tpu as pltpu
from jax.experimental.pallas import tpu_sc as plsc

N = 10000
E = 320000
NC = 2
NT = 16
RPT = 624
NTAIL = N - NT * RPT
CHUNK = 128

_MESH = dict(core_axis_name="c", subcore_axis_name="s")


_ET1 = E // (NC * NT)
_NCH1 = _ET1 // CHUNK
_REM1 = _ET1 - _NCH1 * CHUNK


@functools.partial(
    pl.kernel,
    out_type=jax.ShapeDtypeStruct((NC * N, 16), jnp.float32),
    mesh=plsc.VectorSubcoreMesh(**_MESH),
    scratch_types=[
        pltpu.VMEM_SHARED((N, 16), jnp.float32),
        pltpu.VMEM((CHUNK,), jnp.int32),
        pltpu.VMEM((_REM1,), jnp.int32),
        pltpu.VMEM((CHUNK, 16), jnp.float32),
        pltpu.VMEM((RPT, 16), jnp.float32),
    ],
)
def _deg_sc(dst_hbm, out_hbm, dacc, dst_v, dstT, ones_v, zbuf):
    c = lax.axis_index("c")
    t = lax.axis_index("s")
    one_row = jnp.where(lax.iota(jnp.int32, 16) == 0,
                        jnp.float32(1.0), jnp.float32(0.0))
    zero_row = jnp.zeros((16,), jnp.float32)

    def fill_ones(j, _):
        ones_v[j] = one_row
        return 0

    lax.fori_loop(0, CHUNK, fill_ones, 0)

    def fill_zero(j, _):
        zbuf[j] = zero_row
        return 0

    lax.fori_loop(0, RPT, fill_zero, 0)
    pltpu.sync_copy(zbuf, dacc.at[pl.ds(t * RPT, RPT)])

    @pl.when(t == NT - 1)
    def _():
        pltpu.sync_copy(zbuf.at[pl.ds(0, NTAIL)],
                        dacc.at[pl.ds(NT * RPT, NTAIL)])

    plsc.subcore_barrier()

    base = c * (E // NC) + t * _ET1

    def chunk(i, _):
        pltpu.sync_copy(dst_hbm.at[pl.ds(base + i * CHUNK, CHUNK)], dst_v)
        pltpu.sync_copy(ones_v, dacc.at[dst_v], add=True)
        return 0

    lax.fori_loop(0, _NCH1, chunk, 0)
    pltpu.sync_copy(dst_hbm.at[pl.ds(base + _NCH1 * CHUNK, _REM1)], dstT)
    pltpu.sync_copy(ones_v.at[pl.ds(0, _REM1)], dacc.at[dstT], add=True)
    plsc.subcore_barrier()
    pltpu.sync_copy(dacc.at[pl.ds(t * RPT, RPT)],
                    out_hbm.at[pl.ds(c * N + t * RPT, RPT)])

    @pl.when(t == NT - 1)
    def _():
        pltpu.sync_copy(dacc.at[pl.ds(NT * RPT, NTAIL)],
                        out_hbm.at[pl.ds(c * N + NT * RPT, NTAIL)])


_ET2 = E // NT
_NCH2 = _ET2 // CHUNK
_REM2 = _ET2 - _NCH2 * CHUNK


def _edge_scratch(rem):
    return [
        pltpu.VMEM_SHARED((N, 128), jnp.float32),
        pltpu.VMEM((CHUNK,), jnp.int32),
        pltpu.VMEM((CHUNK,), jnp.int32),
        pltpu.VMEM((CHUNK, 128), jnp.float32),
        pltpu.VMEM((rem,), jnp.int32),
        pltpu.VMEM((rem,), jnp.int32),
        pltpu.VMEM((rem, 128), jnp.float32),
        pltpu.SemaphoreType.DMA,
    ]


@functools.partial(
    pl.kernel,
    out_type=jax.ShapeDtypeStruct((NC * N, 128), jnp.float32),
    mesh=plsc.VectorSubcoreMesh(**_MESH),
    scratch_types=_edge_scratch(_REM1),
)
def _edge_pass_l1(xws_hbm, z_hbm, src_hbm, dst_hbm, out_hbm,
                  acc, src_v, dst_v, rows_v, srcT, dstT, rowsT, sem):
    c = lax.axis_index("c")
    t = lax.axis_index("s")
    r0 = t * RPT

    @pl.when(c == 0)
    def _():
        pltpu.sync_copy(xws_hbm.at[pl.ds(r0, RPT)], acc.at[pl.ds(r0, RPT)])

        @pl.when(t == NT - 1)
        def _():
            pltpu.sync_copy(xws_hbm.at[pl.ds(NT * RPT, NTAIL)],
                            acc.at[pl.ds(NT * RPT, NTAIL)])

    @pl.when(c == 1)
    def _():
        pltpu.sync_copy(z_hbm.at[pl.ds(r0, RPT)], acc.at[pl.ds(r0, RPT)])

        @pl.when(t == NT - 1)
        def _():
            pltpu.sync_copy(z_hbm.at[pl.ds(NT * RPT, NTAIL)],
                            acc.at[pl.ds(NT * RPT, NTAIL)])

    plsc.subcore_barrier()
    base = c * (E // NC) + t * _ET1

    def chunk(i, _):
        eb = base + i * CHUNK
        pltpu.sync_copy(src_hbm.at[pl.ds(eb, CHUNK)], src_v)
        pltpu.sync_copy(dst_hbm.at[pl.ds(eb, CHUNK)], dst_v)
        pltpu.async_copy(xws_hbm.at[src_v], rows_v, sem).wait()
        pltpu.sync_copy(rows_v, acc.at[dst_v], add=True)
        return 0

    lax.fori_loop(0, _NCH1, chunk, 0)
    eb = base + _NCH1 * CHUNK
    pltpu.sync_copy(src_hbm.at[pl.ds(eb, _REM1)], srcT)
    pltpu.sync_copy(dst_hbm.at[pl.ds(eb, _REM1)], dstT)
    pltpu.async_copy(xws_hbm.at[srcT], rowsT, sem).wait()
    pltpu.sync_copy(rowsT, acc.at[dstT], add=True)
    plsc.subcore_barrier()
    pltpu.sync_copy(acc.at[pl.ds(r0, RPT)],
                    out_hbm.at[pl.ds(c * N + r0, RPT)])

    @pl.when(t == NT - 1)
    def _():
        pltpu.sync_copy(acc.at[pl.ds(NT * RPT, NTAIL)],
                        out_hbm.at[pl.ds(c * N + NT * RPT, NTAIL)])


@functools.partial(
    pl.kernel,
    out_type=jax.ShapeDtypeStruct((NC * N, 128), jnp.float32),
    mesh=plsc.VectorSubcoreMesh(**_MESH),
    scratch_types=_edge_scratch(_REM2),
)
def _edge_pass_l2(xws_hbm, src_hbm, dst_hbm, out_hbm,
                  acc, src_v, dst_v, rows_v, srcT, dstT, rowsT, sem):
    c = lax.axis_index("c")
    t = lax.axis_index("s")
    r0 = t * RPT
    cN = c * N
    pltpu.sync_copy(xws_hbm.at[pl.ds(cN + r0, RPT)], acc.at[pl.ds(r0, RPT)])

    @pl.when(t == NT - 1)
    def _():
        pltpu.sync_copy(xws_hbm.at[pl.ds(cN + NT * RPT, NTAIL)],
                        acc.at[pl.ds(NT * RPT, NTAIL)])

    plsc.subcore_barrier()
    base = t * _ET2

    def chunk(i, _):
        eb = base + i * CHUNK
        pltpu.sync_copy(src_hbm.at[pl.ds(eb, CHUNK)], src_v)
        pltpu.sync_copy(dst_hbm.at[pl.ds(eb, CHUNK)], dst_v)
        for k in range(CHUNK // 16):
            src_v[pl.ds(k * 16, 16)] = src_v[pl.ds(k * 16, 16)] + cN
        pltpu.async_copy(xws_hbm.at[src_v], rows_v, sem).wait()
        pltpu.sync_copy(rows_v, acc.at[dst_v], add=True)
        return 0

    lax.fori_loop(0, _NCH2, chunk, 0)
    eb = base + _NCH2 * CHUNK
    pltpu.sync_copy(src_hbm.at[pl.ds(eb, _REM2)], srcT)
    pltpu.sync_copy(dst_hbm.at[pl.ds(eb, _REM2)], dstT)
    for k in range(_REM2 // 16):
        srcT[pl.ds(k * 16, 16)] = srcT[pl.ds(k * 16, 16)] + cN
    pltpu.async_copy(xws_hbm.at[srcT], rowsT, sem).wait()
    pltpu.sync_copy(rowsT, acc.at[dstT], add=True)
    plsc.subcore_barrier()
    pltpu.sync_copy(acc.at[pl.ds(r0, RPT)],
                    out_hbm.at[pl.ds(cN + r0, RPT)])

    @pl.when(t == NT - 1)
    def _():
        pltpu.sync_copy(acc.at[pl.ds(NT * RPT, NTAIL)],
                        out_hbm.at[pl.ds(cN + NT * RPT, NTAIL)])


_JPT = 1024 // (NC * NT)


@functools.partial(
    pl.kernel,
    out_type=jax.ShapeDtypeStruct((1024,), jnp.float32),
    mesh=plsc.VectorSubcoreMesh(**_MESH),
    scratch_types=[
        pltpu.VMEM((N,), jnp.float32),
        pltpu.VMEM((_JPT,), jnp.int32),
        pltpu.VMEM((_JPT,), jnp.float32),
    ],
    compiler_params=pltpu.CompilerParams(needs_layout_passes=False),
)
def _join_sc(y_hbm, join_hbm, out_hbm, ybuf, jv, res):
    c = lax.axis_index("c")
    t = lax.axis_index("s")
    wid = t * NC + c
    pltpu.sync_copy(y_hbm, ybuf)
    pltpu.sync_copy(join_hbm.at[pl.ds(wid * _JPT, _JPT)], jv)
    for k in range(_JPT // 16):
        idx = jv[pl.ds(k * 16, 16)]
        v = plsc.load_gather(ybuf, [idx])
        res[pl.ds(k * 16, 16)] = 1.0 / (1.0 + jnp.exp(-v))
    pltpu.sync_copy(res, out_hbm.at[pl.ds(wid * _JPT, _JPT)])


_BR = 1000
_GR = N // _BR


def _dinv_of(deg_blk):
    deg = deg_blk[0, :, 0] + deg_blk[1, :, 0] + 1.0
    return lax.rsqrt(deg)


def _mm1_body(x_ref, w1_ref, deg_ref, o_ref):
    dinv = _dinv_of(deg_ref[...])
    xw = jnp.dot(x_ref[...], w1_ref[...], preferred_element_type=jnp.float32)
    o_ref[...] = xw * dinv[:, None]


def _mm2_body(o1_ref, deg_ref, b1_ref, w2_ref, o_ref):
    dinv = _dinv_of(deg_ref[...])
    h = (o1_ref[0] + o1_ref[1]) * dinv[:, None]
    h = jnp.maximum(h + b1_ref[0], 0.0)
    xw = jnp.dot(h, w2_ref[...], preferred_element_type=jnp.float32)
    o_ref[...] = xw * dinv[:, None]


def _mm3_body(o2_ref, deg_ref, b2_ref, w3_ref, b3_ref, y_ref):
    dinv = _dinv_of(deg_ref[...])
    h = jnp.concatenate([o2_ref[0], o2_ref[1]], axis=1) * dinv[:, None]
    h = h + b2_ref[0]
    y = jnp.dot(h, w3_ref[...], preferred_element_type=jnp.float32)
    y_ref[...] = y + b3_ref[0, 0]


def _mm1(x, w1, deg2):
    return pl.pallas_call(
        _mm1_body,
        grid=(_GR,),
        in_specs=[
            pl.BlockSpec((_BR, 128), lambda i: (i, 0)),
            pl.BlockSpec((128, 128), lambda i: (0, 0)),
            pl.BlockSpec((NC, _BR, 16), lambda i: (0, i, 0)),
        ],
        out_specs=pl.BlockSpec((_BR, 128), lambda i: (i, 0)),
        out_shape=jax.ShapeDtypeStruct((N, 128), jnp.float32),
    )(x, w1, deg2)


def _mm2(out1, deg2, b1, w2):
    return pl.pallas_call(
        _mm2_body,
        grid=(_GR, NC),
        in_specs=[
            pl.BlockSpec((NC, _BR, 128), lambda i, c: (0, i, 0)),
            pl.BlockSpec((NC, _BR, 16), lambda i, c: (0, i, 0)),
            pl.BlockSpec((1, 128), lambda i, c: (0, 0)),
            pl.BlockSpec((128, 128), lambda i, c: (0, c)),
        ],
        out_specs=pl.BlockSpec((_BR, 128), lambda i, c: (c * _GR + i, 0)),
        out_shape=jax.ShapeDtypeStruct((NC * N, 128), jnp.float32),
    )(out1, deg2, b1, w2)


def _mm3(out2, deg2, b2, w3, b3):
    return pl.pallas_call(
        _mm3_body,
        grid=(_GR,),
        in_specs=[
            pl.BlockSpec((NC, _BR, 128), lambda i: (0, i, 0)),
            pl.BlockSpec((NC, _BR, 16), lambda i: (0, i, 0)),
            pl.BlockSpec((1, 256), lambda i: (0, 0)),
            pl.BlockSpec((256, 1), lambda i: (0, 0)),
            pl.BlockSpec((1, 1), lambda i: (0, 0)),
        ],
        out_specs=pl.BlockSpec((_BR, 1), lambda i: (i, 0)),
        out_shape=jax.ShapeDtypeStruct((N, 1), jnp.float32),
    )(out2, deg2, b2, w3, b3)


def kernel(x, edge_index, join_index, W1, b1, W2, b2, W3, b3):
    src = edge_index[0].astype(jnp.int32)
    dst = edge_index[1].astype(jnp.int32)
    join = join_index.astype(jnp.int32)

    deg2 = _deg_sc(dst).reshape(NC, N, 16)
    xws1 = _mm1(x, W1, deg2)
    zeros = jnp.zeros((N, 128), jnp.float32)
    out1 = _edge_pass_l1(xws1, zeros, src, dst).reshape(NC, N, 128)
    xws2 = _mm2(out1, deg2, b1.reshape(1, 128), W2)
    out2 = _edge_pass_l2(xws2, src, dst).reshape(NC, N, 128)
    y = _mm3(out2, deg2, b2.reshape(1, 256), W3, b3.reshape(1, 1))
    z = _join_sc(y.reshape(N), join)
    return z.reshape(1024, 1)
